# baseline (speedup 1.0000x reference)
import functools

import jax
import jax.numpy as jnp
from jax import lax
from jax.experimental import pallas as pl
from jax.experimental.pallas import tpu as pltpu
from jax.experimental.pallas import tpu_sc as plsc

NC = 2
NS = 16
NW = NC * NS
CH = 128
DW = 16


def _mesh():
    return plsc.VectorSubcoreMesh(
        core_axis_name="c", subcore_axis_name="s",
        num_cores=NC, num_subcores=NS)


NBUF = 4
BG = 8
SPLIT0 = 0.25


@functools.lru_cache(maxsize=None)
def _sc_deg(np_, ep):
    rpt = np_ // NS
    ept = ep // NW
    nch = ept // CH

    @functools.partial(
        pl.kernel,
        out_type=jax.ShapeDtypeStruct((NC, NS, rpt, DW), jnp.float32),
        mesh=_mesh(),
        scratch_types=[
            pltpu.VMEM((16, DW), jnp.float32),
            pltpu.VMEM((CH, DW), jnp.float32),
            pltpu.VMEM((nch, CH), jnp.int32),
            pltpu.VMEM_SHARED((np_, DW), jnp.float32),
            pltpu.SemaphoreType.DMA,
        ] + [pltpu.SemaphoreType.DMA for _ in range(NBUF)],
    )
    def body(dst2, out, zbuf, ones, dall, acc, zsem, *ssem):
        cid = lax.axis_index("c")
        sid = lax.axis_index("s")
        wid = sid * NC + cid
        z16 = jnp.zeros((16,), jnp.float32)
        o16 = jnp.ones((16,), jnp.float32)
        for i in range(16):
            zbuf[i, pl.ds(0, 16)] = z16
        for i in range(CH):
            ones[i, pl.ds(0, 16)] = o16
        for k in range(rpt // 16):
            pltpu.async_copy(zbuf, acc.at[pl.ds(sid * rpt + k * 16, 16)], zsem)
        pltpu.sync_copy(dst2.at[pl.ds(wid * nch, nch)], dall)
        for k in range(rpt // 16):
            pltpu.make_async_copy(
                zbuf, acc.at[pl.ds(sid * rpt + k * 16, 16)], zsem).wait()
        plsc.subcore_barrier()

        def scat(i, b):
            pltpu.async_copy(ones, acc.at[dall.at[i]], ssem[b], add=True)

        def sdrain(b):
            pltpu.make_async_copy(ones, acc.at[dall.at[0]], ssem[b]).wait()

        for b in range(NBUF):
            scat(b, b)

        def eloop(g, c):
            for b in range(NBUF):
                sdrain(b)
                scat(g * NBUF + b, b)
            return c
        lax.fori_loop(1, nch // NBUF, eloop, 0)
        for b in range(NBUF):
            sdrain(b)
        plsc.subcore_barrier()
        pltpu.sync_copy(acc.at[pl.ds(sid * rpt, rpt)], out.at[cid, sid])

    return body


@functools.lru_cache(maxsize=None)
def _sc_scatter(np_, d, ep):
    rpt = np_ // NS
    ept = ep // NW
    nch = ept // CH

    ncht = ep // CH
    unit = 2 * BG
    nctot = ncht // NS
    nch0 = max(unit, min(nctot - unit,
                         int(round(nctot * SPLIT0 / unit)) * unit))
    nch1 = nctot - nch0

    @functools.partial(
        pl.kernel,
        out_type=jax.ShapeDtypeStruct((NC, NS, rpt, d), jnp.float32),
        mesh=_mesh(),
        scratch_types=[
            pltpu.VMEM((16, d), jnp.float32),
            pltpu.VMEM((BG, CH), jnp.int32),
            pltpu.VMEM((BG, CH), jnp.int32),
            pltpu.VMEM((BG, CH), jnp.int32),
            pltpu.VMEM((BG, CH), jnp.int32),
            pltpu.VMEM((CH, d), jnp.float32),
            pltpu.VMEM((CH, d), jnp.float32),
            pltpu.VMEM_SHARED((np_, d), jnp.float32),
        ] + [pltpu.SemaphoreType.DMA for _ in range(7)],
    )
    def body(hs, src2, dst2, out, zbuf, sb0, sb1, db0, db1, r0, r1, acc,
             zsem, i0, i1, g0, g1, s0, s1):
        sblk, dblk, rows = [sb0, sb1], [db0, db1], [r0, r1]
        isem, gsem, ssem = [i0, i1], [g0, g1], [s0, s1]
        cid = lax.axis_index("c")
        sid = lax.axis_index("s")
        cbase = jnp.where(cid == 0, sid * nch0, NS * nch0 + sid * nch1)
        z16 = jnp.zeros((16,), jnp.float32)
        for i in range(16):
            for j in range(d // 16):
                zbuf[i, pl.ds(j * 16, 16)] = z16
        for k in range(rpt // 16):
            pltpu.async_copy(zbuf, acc.at[pl.ds(sid * rpt + k * 16, 16)], zsem)

        def ifetch(kb, p):
            off = pl.multiple_of(cbase + kb * BG, 8)
            pltpu.async_copy(src2.at[pl.ds(off, BG)], sblk[p], isem[p])
            pltpu.async_copy(dst2.at[pl.ds(off, BG)], dblk[p], isem[p])

        def idrain(p):
            pltpu.make_async_copy(src2.at[pl.ds(0, BG)], sblk[p],
                                  isem[p]).wait()
            pltpu.make_async_copy(dst2.at[pl.ds(0, BG)], dblk[p],
                                  isem[p]).wait()

        def block(p, kb_next):
            def gath(j, b):
                pltpu.async_copy(hs.at[sblk[p].at[j]], rows[b], gsem[b])

            def gdrain(b):
                pltpu.make_async_copy(hs.at[sblk[p].at[0]], rows[b],
                                      gsem[b]).wait()

            def scat(j, b):
                pltpu.async_copy(rows[b], acc.at[dblk[p].at[j]], ssem[b],
                                 add=True)

            def sdrain(b):
                pltpu.make_async_copy(rows[b], acc.at[dblk[p].at[0]],
                                      ssem[b]).wait()

            idrain(p)
            gath(0, 0)
            gath(1, 1)
            if kb_next is not None:
                ifetch(kb_next, 1 - p)
            for j in range(BG):
                b = j % 2
                gdrain(b)
                scat(j, b)
                if j + 2 < BG:
                    sdrain(b)
                    gath(j + 2, b)
            sdrain(0)
            sdrain(1)

        ifetch(0, 0)
        for k in range(rpt // 16):
            pltpu.make_async_copy(
                zbuf, acc.at[pl.ds(sid * rpt + k * 16, 16)], zsem).wait()
        plsc.subcore_barrier()

        def pair(kp, c):
            block(0, kp * 2 + 1)
            block(1, kp * 2 + 2)
            return c

        def run(npair_s):
            lax.fori_loop(0, npair_s - 1, pair, 0)
            block(0, 2 * npair_s - 1)
            block(1, None)

        @pl.when(cid == 0)
        def _():
            run(nch0 // unit)

        @pl.when(cid != 0)
        def _():
            run(nch1 // unit)
        plsc.subcore_barrier()
        pltpu.sync_copy(acc.at[pl.ds(sid * rpt, rpt)], out.at[cid, sid])

    return body


@functools.lru_cache(maxsize=None)
def _tc_dinv(np_):
    def body(deg_ref, out_ref):
        d = deg_ref[0] + deg_ref[1] + 1.0
        out_ref[...] = lax.rsqrt(d)
    return pl.pallas_call(
        body,
        out_shape=jax.ShapeDtypeStruct((np_, DW), jnp.float32),
    )


@functools.lru_cache(maxsize=None)
def _tc_mm_scale(np_, d, br):
    grid = (np_ // br,)

    def body(x_ref, w_ref, dinv_ref, out_ref):
        h = jnp.dot(x_ref[...], w_ref[...],
                    preferred_element_type=jnp.float32)
        out_ref[...] = h * dinv_ref[...]

    return pl.pallas_call(
        body,
        grid=grid,
        in_specs=[
            pl.BlockSpec((br, d), lambda i: (i, 0)),
            pl.BlockSpec((d, d), lambda i: (0, 0)),
            pl.BlockSpec((br, 1), lambda i: (i, 0)),
        ],
        out_specs=pl.BlockSpec((br, d), lambda i: (i, 0)),
        out_shape=jax.ShapeDtypeStruct((np_, d), jnp.float32),
    )


@functools.lru_cache(maxsize=None)
def _tc_layer(np_, d, n, br):
    grid = (np_ // br,)

    def body(p0_ref, p1_ref, hs_ref, dinv_ref, b_ref, w_ref, out_ref):
        dinv = dinv_ref[...]
        a = (p0_ref[...] + p1_ref[...] + hs_ref[...]) * dinv + b_ref[...]
        rows = (lax.broadcasted_iota(jnp.int32, (br, 1), 0)
                + pl.program_id(0) * br)
        a = jnp.where(rows < n, jnp.maximum(a, 0.0), 0.0)
        out_ref[...] = jnp.dot(a, w_ref[...],
                               preferred_element_type=jnp.float32) * dinv

    return pl.pallas_call(
        body,
        grid=grid,
        in_specs=[
            pl.BlockSpec((br, d), lambda i: (i, 0)),
            pl.BlockSpec((br, d), lambda i: (i, 0)),
            pl.BlockSpec((br, d), lambda i: (i, 0)),
            pl.BlockSpec((br, 1), lambda i: (i, 0)),
            pl.BlockSpec((1, d), lambda i: (0, 0)),
            pl.BlockSpec((d, d), lambda i: (0, 0)),
        ],
        out_specs=pl.BlockSpec((br, d), lambda i: (i, 0)),
        out_shape=jax.ShapeDtypeStruct((np_, d), jnp.float32),
    )


@functools.lru_cache(maxsize=None)
def _tc_final(np_, d, br):
    grid = (np_ // br,)

    def body(p0_ref, p1_ref, hs_ref, dinv_ref, b_ref, out_ref):
        out_ref[...] = ((p0_ref[...] + p1_ref[...] + hs_ref[...])
                        * dinv_ref[...] + b_ref[...])

    return pl.pallas_call(
        body,
        grid=grid,
        in_specs=[
            pl.BlockSpec((br, d), lambda i: (i, 0)),
            pl.BlockSpec((br, d), lambda i: (i, 0)),
            pl.BlockSpec((br, d), lambda i: (i, 0)),
            pl.BlockSpec((br, 1), lambda i: (i, 0)),
            pl.BlockSpec((1, d), lambda i: (0, 0)),
        ],
        out_specs=pl.BlockSpec((br, d), lambda i: (i, 0)),
        out_shape=jax.ShapeDtypeStruct((np_, d), jnp.float32),
    )


def kernel(x, edge_index, batch, W1, b1, W2, b2, W3, b3):
    n, d = x.shape
    e = edge_index.shape[1]
    np_ = ((n + 1 + 127) // 128) * 128
    eb = NW * CH * 2 * BG
    ep = ((e + eb - 1) // eb) * eb
    nch = ep // (NW * CH)
    br = 1280 if np_ % 1280 == 0 else 128

    pad = ep - e
    src = edge_index[0]
    dst = edge_index[1]
    if pad:
        fill = jnp.full((pad,), n, jnp.int32)
        src = jnp.concatenate([src, fill])
        dst = jnp.concatenate([dst, fill])
    src = src.reshape(ep // CH, CH)
    dst = dst.reshape(ep // CH, CH)
    x_pad = jnp.pad(x, ((0, np_ - n), (0, 0)))

    degp = _sc_deg(np_, ep)(dst)
    dinv16 = _tc_dinv(np_)(degp.reshape(NC, np_, DW))
    dinv_col = dinv16[:, :1]

    hs = _tc_mm_scale(np_, d, br)(x_pad, W1, dinv_col)
    scat = _sc_scatter(np_, d, ep)
    layer = _tc_layer(np_, d, n, br)
    for w_next, b_cur in ((W2, b1), (W3, b2)):
        p = scat(hs, src, dst)
        hs = layer(p[0].reshape(np_, d), p[1].reshape(np_, d),
                   hs, dinv_col, b_cur.reshape(1, d), w_next)
    p = scat(hs, src, dst)
    out = _tc_final(np_, d, br)(p[0].reshape(np_, d), p[1].reshape(np_, d),
                                hs, dinv_col, b3.reshape(1, d))
    return out[:n]


# speedup vs baseline: 1.0459x; 1.0459x over previous
import functools

import jax
import jax.numpy as jnp
from jax import lax
from jax.experimental import pallas as pl
from jax.experimental.pallas import tpu as pltpu
from jax.experimental.pallas import tpu_sc as plsc

NC = 2
NS = 16
NW = NC * NS
CH = 128
DW = 16


def _mesh():
    return plsc.VectorSubcoreMesh(
        core_axis_name="c", subcore_axis_name="s",
        num_cores=NC, num_subcores=NS)


NBUF = 4
BG = 8
SPLIT0 = 0.5


@functools.lru_cache(maxsize=None)
def _sc_deg(np_, ep):
    rpt = np_ // NS
    ept = ep // NW
    nch = ept // CH

    @functools.partial(
        pl.kernel,
        out_type=jax.ShapeDtypeStruct((NC, NS, rpt, DW), jnp.float32),
        mesh=_mesh(),
        scratch_types=[
            pltpu.VMEM((16, DW), jnp.float32),
            pltpu.VMEM((CH, DW), jnp.float32),
            pltpu.VMEM((nch, CH), jnp.int32),
            pltpu.VMEM_SHARED((np_, DW), jnp.float32),
            pltpu.SemaphoreType.DMA,
        ] + [pltpu.SemaphoreType.DMA for _ in range(NBUF)],
    )
    def body(dst2, out, zbuf, ones, dall, acc, zsem, *ssem):
        cid = lax.axis_index("c")
        sid = lax.axis_index("s")
        wid = sid * NC + cid
        z16 = jnp.zeros((16,), jnp.float32)
        o16 = jnp.ones((16,), jnp.float32)
        for i in range(16):
            zbuf[i, pl.ds(0, 16)] = z16
        for i in range(CH):
            ones[i, pl.ds(0, 16)] = o16
        for k in range(rpt // 16):
            pltpu.async_copy(zbuf, acc.at[pl.ds(sid * rpt + k * 16, 16)], zsem)
        pltpu.sync_copy(dst2.at[pl.ds(wid * nch, nch)], dall)
        for k in range(rpt // 16):
            pltpu.make_async_copy(
                zbuf, acc.at[pl.ds(sid * rpt + k * 16, 16)], zsem).wait()
        plsc.subcore_barrier()

        def scat(i, b):
            pltpu.async_copy(ones, acc.at[dall.at[i]], ssem[b], add=True)

        def sdrain(b):
            pltpu.make_async_copy(ones, acc.at[dall.at[0]], ssem[b]).wait()

        for b in range(NBUF):
            scat(b, b)

        def eloop(g, c):
            for b in range(NBUF):
                sdrain(b)
                scat(g * NBUF + b, b)
            return c
        lax.fori_loop(1, nch // NBUF, eloop, 0)
        for b in range(NBUF):
            sdrain(b)
        plsc.subcore_barrier()
        pltpu.sync_copy(acc.at[pl.ds(sid * rpt, rpt)], out.at[cid, sid])

    return body


@functools.lru_cache(maxsize=None)
def _sc_scatter(np_, d, ep):
    rpt = np_ // NS
    ept = ep // NW
    nch = ept // CH

    ncht = ep // CH
    unit = 2 * BG
    nctot = ncht // NS
    nch0 = max(unit, min(nctot - unit,
                         int(round(nctot * SPLIT0 / unit)) * unit))
    nch1 = nctot - nch0

    @functools.partial(
        pl.kernel,
        out_type=jax.ShapeDtypeStruct((NC, NS, rpt, d), jnp.float32),
        mesh=_mesh(),
        scratch_types=[
            pltpu.VMEM((16, d), jnp.float32),
            pltpu.VMEM((BG, CH), jnp.int32),
            pltpu.VMEM((BG, CH), jnp.int32),
            pltpu.VMEM((BG, CH), jnp.int32),
            pltpu.VMEM((BG, CH), jnp.int32),
            pltpu.VMEM((CH, d), jnp.float32),
            pltpu.VMEM((CH, d), jnp.float32),
            pltpu.VMEM_SHARED((np_, d), jnp.float32),
        ] + [pltpu.SemaphoreType.DMA for _ in range(7)],
    )
    def body(hs, src2, dst2, out, zbuf, sb0, sb1, db0, db1, r0, r1, acc,
             zsem, i0, i1, g0, g1, s0, s1):
        sblk, dblk, rows = [sb0, sb1], [db0, db1], [r0, r1]
        isem, gsem, ssem = [i0, i1], [g0, g1], [s0, s1]
        cid = lax.axis_index("c")
        sid = lax.axis_index("s")
        cbase = jnp.where(cid == 0, sid * nch0, NS * nch0 + sid * nch1)
        z16 = jnp.zeros((16,), jnp.float32)
        for i in range(16):
            for j in range(d // 16):
                zbuf[i, pl.ds(j * 16, 16)] = z16
        for k in range(rpt // 16):
            pltpu.async_copy(zbuf, acc.at[pl.ds(sid * rpt + k * 16, 16)], zsem)

        def ifetch(kb, p):
            off = pl.multiple_of(cbase + kb * BG, 8)
            pltpu.async_copy(src2.at[pl.ds(off, BG)], sblk[p], isem[p])
            pltpu.async_copy(dst2.at[pl.ds(off, BG)], dblk[p], isem[p])

        def idrain(p):
            pltpu.make_async_copy(src2.at[pl.ds(0, BG)], sblk[p],
                                  isem[p]).wait()
            pltpu.make_async_copy(dst2.at[pl.ds(0, BG)], dblk[p],
                                  isem[p]).wait()

        def block(p, kb_next):
            def gath(j, b):
                pltpu.async_copy(hs.at[sblk[p].at[j]], rows[b], gsem[b])

            def gdrain(b):
                pltpu.make_async_copy(hs.at[sblk[p].at[0]], rows[b],
                                      gsem[b]).wait()

            def scat(j, b):
                pltpu.async_copy(rows[b], acc.at[dblk[p].at[j]], ssem[b],
                                 add=True)

            def sdrain(b):
                pltpu.make_async_copy(rows[b], acc.at[dblk[p].at[0]],
                                      ssem[b]).wait()

            idrain(p)
            gath(0, 0)
            gath(1, 1)
            if kb_next is not None:
                ifetch(kb_next, 1 - p)
            for j in range(BG):
                b = j % 2
                gdrain(b)
                scat(j, b)
                if j + 2 < BG:
                    sdrain(b)
                    gath(j + 2, b)
            sdrain(0)
            sdrain(1)

        ifetch(0, 0)
        for k in range(rpt // 16):
            pltpu.make_async_copy(
                zbuf, acc.at[pl.ds(sid * rpt + k * 16, 16)], zsem).wait()
        plsc.subcore_barrier()

        def pair(kp, c):
            block(0, kp * 2 + 1)
            block(1, kp * 2 + 2)
            return c

        def run(npair_s):
            lax.fori_loop(0, npair_s - 1, pair, 0)
            block(0, 2 * npair_s - 1)
            block(1, None)

        @pl.when(cid == 0)
        def _():
            run(nch0 // unit)

        @pl.when(cid != 0)
        def _():
            run(nch1 // unit)
        plsc.subcore_barrier()
        pltpu.sync_copy(acc.at[pl.ds(sid * rpt, rpt)], out.at[cid, sid])

    return body


@functools.lru_cache(maxsize=None)
def _tc_dinv(np_, d, br):
    grid = (np_ // br,)

    def body(deg_ref, out_ref):
        s = lax.rsqrt(deg_ref[0] + deg_ref[1] + 1.0)
        out_ref[...] = jnp.broadcast_to(s[:, :1], (br, d))

    return pl.pallas_call(
        body,
        grid=grid,
        in_specs=[pl.BlockSpec((NC, br, DW), lambda i: (0, i, 0))],
        out_specs=pl.BlockSpec((br, d), lambda i: (i, 0)),
        out_shape=jax.ShapeDtypeStruct((np_, d), jnp.float32),
    )


@functools.lru_cache(maxsize=None)
def _tc_mm_scale(np_, d, br):
    grid = (np_ // br,)

    def body(x_ref, w_ref, dinv_ref, out_ref):
        h = jnp.dot(x_ref[...], w_ref[...],
                    preferred_element_type=jnp.float32)
        out_ref[...] = h * dinv_ref[...]

    return pl.pallas_call(
        body,
        grid=grid,
        in_specs=[
            pl.BlockSpec((br, d), lambda i: (i, 0)),
            pl.BlockSpec((d, d), lambda i: (0, 0)),
            pl.BlockSpec((br, d), lambda i: (i, 0)),
        ],
        out_specs=pl.BlockSpec((br, d), lambda i: (i, 0)),
        out_shape=jax.ShapeDtypeStruct((np_, d), jnp.float32),
    )


@functools.lru_cache(maxsize=None)
def _tc_layer(np_, d, n, br):
    grid = (np_ // br,)

    def body(p0_ref, p1_ref, hs_ref, dinv_ref, b_ref, w_ref, out_ref):
        dinv = dinv_ref[...]
        a = (p0_ref[...] + p1_ref[...] + hs_ref[...]) * dinv + b_ref[...]
        rows = (lax.broadcasted_iota(jnp.int32, (br, 1), 0)
                + pl.program_id(0) * br)
        a = jnp.where(rows < n, jnp.maximum(a, 0.0), 0.0)
        out_ref[...] = jnp.dot(a, w_ref[...],
                               preferred_element_type=jnp.float32) * dinv

    return pl.pallas_call(
        body,
        grid=grid,
        in_specs=[
            pl.BlockSpec((br, d), lambda i: (i, 0)),
            pl.BlockSpec((br, d), lambda i: (i, 0)),
            pl.BlockSpec((br, d), lambda i: (i, 0)),
            pl.BlockSpec((br, d), lambda i: (i, 0)),
            pl.BlockSpec((1, d), lambda i: (0, 0)),
            pl.BlockSpec((d, d), lambda i: (0, 0)),
        ],
        out_specs=pl.BlockSpec((br, d), lambda i: (i, 0)),
        out_shape=jax.ShapeDtypeStruct((np_, d), jnp.float32),
    )


@functools.lru_cache(maxsize=None)
def _tc_final(np_, d, br):
    grid = (np_ // br,)

    def body(p0_ref, p1_ref, hs_ref, dinv_ref, b_ref, out_ref):
        out_ref[...] = ((p0_ref[...] + p1_ref[...] + hs_ref[...])
                        * dinv_ref[...] + b_ref[...])

    return pl.pallas_call(
        body,
        grid=grid,
        in_specs=[
            pl.BlockSpec((br, d), lambda i: (i, 0)),
            pl.BlockSpec((br, d), lambda i: (i, 0)),
            pl.BlockSpec((br, d), lambda i: (i, 0)),
            pl.BlockSpec((br, d), lambda i: (i, 0)),
            pl.BlockSpec((1, d), lambda i: (0, 0)),
        ],
        out_specs=pl.BlockSpec((br, d), lambda i: (i, 0)),
        out_shape=jax.ShapeDtypeStruct((np_, d), jnp.float32),
    )


def kernel(x, edge_index, batch, W1, b1, W2, b2, W3, b3):
    n, d = x.shape
    e = edge_index.shape[1]
    np_ = ((n + 1 + 127) // 128) * 128
    eb = NW * CH * 2 * BG
    ep = ((e + eb - 1) // eb) * eb
    nch = ep // (NW * CH)
    br = 1280 if np_ % 1280 == 0 else 128

    pad = ep - e
    src = edge_index[0]
    dst = edge_index[1]
    if pad:
        fill = jnp.full((pad,), n, jnp.int32)
        src = jnp.concatenate([src, fill])
        dst = jnp.concatenate([dst, fill])
    src = src.reshape(ep // CH, CH)
    dst = dst.reshape(ep // CH, CH)
    x_pad = jnp.pad(x, ((0, np_ - n), (0, 0)))

    degp = _sc_deg(np_, ep)(dst)
    dinvb = _tc_dinv(np_, d, br)(degp.reshape(NC, np_, DW))

    hs = _tc_mm_scale(np_, d, br)(x_pad, W1, dinvb)
    scat = _sc_scatter(np_, d, ep)
    layer = _tc_layer(np_, d, n, br)
    for w_next, b_cur in ((W2, b1), (W3, b2)):
        p = scat(hs, src, dst)
        hs = layer(p[0].reshape(np_, d), p[1].reshape(np_, d),
                   hs, dinvb, b_cur.reshape(1, d), w_next)
    p = scat(hs, src, dst)
    out = _tc_final(np_, d, br)(p[0].reshape(np_, d), p[1].reshape(np_, d),
                                hs, dinvb, b3.reshape(1, d))
    return out[:n]


# speedup vs baseline: 1.0655x; 1.0187x over previous
import functools

import jax
import jax.numpy as jnp
from jax import lax
from jax.experimental import pallas as pl
from jax.experimental.pallas import tpu as pltpu
from jax.experimental.pallas import tpu_sc as plsc

NC = 2
NS = 16
NW = NC * NS
CH = 128
DW = 16
NBUF = 4
BG = 8


def _mesh():
    return plsc.VectorSubcoreMesh(
        core_axis_name="c", subcore_axis_name="s",
        num_cores=NC, num_subcores=NS)


@functools.lru_cache(maxsize=None)
def _sc_deg(np_, ep):
    rpt = np_ // NS
    nch = ep // CH // NW

    @functools.partial(
        pl.kernel,
        out_type=jax.ShapeDtypeStruct((NC, NS, rpt, DW), jnp.float32),
        mesh=_mesh(),
        scratch_types=[
            pltpu.VMEM((16, DW), jnp.float32),
            pltpu.VMEM((CH, DW), jnp.float32),
            pltpu.VMEM((nch, CH), jnp.int32),
            pltpu.VMEM_SHARED((np_, DW), jnp.float32),
            pltpu.SemaphoreType.DMA,
        ] + [pltpu.SemaphoreType.DMA for _ in range(NBUF)],
    )
    def body(dst2, out, zbuf, ones, dall, acc, zsem, *ssem):
        cid = lax.axis_index("c")
        sid = lax.axis_index("s")
        wid = sid * NC + cid
        z16 = jnp.zeros((16,), jnp.float32)
        o16 = jnp.ones((16,), jnp.float32)
        for i in range(16):
            zbuf[i, pl.ds(0, 16)] = z16
        for i in range(CH):
            ones[i, pl.ds(0, 16)] = o16
        for k in range(rpt // 16):
            pltpu.async_copy(zbuf, acc.at[pl.ds(sid * rpt + k * 16, 16)], zsem)
        pltpu.sync_copy(dst2.at[pl.ds(wid * nch, nch)], dall)
        for k in range(rpt // 16):
            pltpu.make_async_copy(
                zbuf, acc.at[pl.ds(sid * rpt + k * 16, 16)], zsem).wait()
        plsc.subcore_barrier()

        def scat(i, b):
            pltpu.async_copy(ones, acc.at[dall.at[i]], ssem[b], add=True)

        def sdrain(b):
            pltpu.make_async_copy(ones, acc.at[dall.at[0]], ssem[b]).wait()

        for b in range(NBUF):
            scat(b, b)

        def eloop(g, c):
            for b in range(NBUF):
                sdrain(b)
                scat(g * NBUF + b, b)
            return c
        lax.fori_loop(1, nch // NBUF, eloop, 0)
        for b in range(NBUF):
            sdrain(b)
        plsc.subcore_barrier()
        pltpu.sync_copy(acc.at[pl.ds(sid * rpt, rpt)], out.at[cid, sid])

    return body


SPLIT0 = 0.5


@functools.lru_cache(maxsize=None)
def _sc_scatter(np_, d, ep):
    rpt = np_ // NS
    ncht = ep // CH
    unit = 2 * BG
    nctot = ncht // NS
    nch0 = max(unit, min(nctot - unit,
                         int(round(nctot * SPLIT0 / unit)) * unit))
    nch1 = nctot - nch0

    @functools.partial(
        pl.kernel,
        out_type=jax.ShapeDtypeStruct((NC, NS, rpt, d), jnp.float32),
        mesh=_mesh(),
        scratch_types=[
            pltpu.VMEM((16, d), jnp.float32),
            pltpu.VMEM((BG, CH), jnp.int32),
            pltpu.VMEM((BG, CH), jnp.int32),
            pltpu.VMEM((BG, CH), jnp.int32),
            pltpu.VMEM((BG, CH), jnp.int32),
            pltpu.VMEM((CH, d), jnp.float32),
            pltpu.VMEM((CH, d), jnp.float32),
            pltpu.VMEM_SHARED((np_, d), jnp.float32),
        ] + [pltpu.SemaphoreType.DMA for _ in range(7)],
    )
    def body(hs, src2, dst2, out, zbuf, sb0, sb1, db0, db1, r0, r1,
             acc, zsem, i0, i1, g0, g1, s0, s1):
        sblk, dblk, rows = [sb0, sb1], [db0, db1], [r0, r1]
        isem, gsem, ssem = [i0, i1], [g0, g1], [s0, s1]
        cid = lax.axis_index("c")
        sid = lax.axis_index("s")
        cbase = jnp.where(cid == 0, sid * nch0, NS * nch0 + sid * nch1)
        z16 = jnp.zeros((16,), jnp.float32)
        for i in range(16):
            for j in range(d // 16):
                zbuf[i, pl.ds(j * 16, 16)] = z16
        for k in range(rpt // 16):
            pltpu.async_copy(zbuf, acc.at[pl.ds(sid * rpt + k * 16, 16)], zsem)

        rsl = pl.ds(sid * rpt, rpt)

        def ifetch(kb, p):
            off = pl.multiple_of(cbase + kb * BG, 8)
            pltpu.async_copy(src2.at[pl.ds(off, BG)], sblk[p], isem[p])
            pltpu.async_copy(dst2.at[pl.ds(off, BG)], dblk[p], isem[p])

        def idrain(p):
            pltpu.make_async_copy(src2.at[pl.ds(0, BG)], sblk[p],
                                  isem[p]).wait()
            pltpu.make_async_copy(dst2.at[pl.ds(0, BG)], dblk[p],
                                  isem[p]).wait()

        def block(p, kb_next):
            def gath(j, b):
                pltpu.async_copy(hs.at[sblk[p].at[j]], rows[b], gsem[b])

            def gdrain(b):
                pltpu.make_async_copy(hs.at[sblk[p].at[0]], rows[b],
                                      gsem[b]).wait()

            def scat(j, b):
                pltpu.async_copy(rows[b], acc.at[dblk[p].at[j]], ssem[b],
                                 add=True)

            def sdrain(b):
                pltpu.make_async_copy(rows[b], acc.at[dblk[p].at[0]],
                                      ssem[b]).wait()

            idrain(p)
            gath(0, 0)
            gath(1, 1)
            if kb_next is not None:
                ifetch(kb_next, 1 - p)
            for j in range(BG):
                b = j % 2
                gdrain(b)
                scat(j, b)
                if j + 2 < BG:
                    sdrain(b)
                    gath(j + 2, b)
            sdrain(0)
            sdrain(1)

        ifetch(0, 0)
        for k in range(rpt // 16):
            pltpu.make_async_copy(
                zbuf, acc.at[pl.ds(sid * rpt + k * 16, 16)], zsem).wait()
        plsc.subcore_barrier()

        def pair(kp, c):
            block(0, kp * 2 + 1)
            block(1, kp * 2 + 2)
            return c

        def run(npair_s):
            lax.fori_loop(0, npair_s - 1, pair, 0)
            block(0, 2 * npair_s - 1)
            block(1, None)

        @pl.when(cid == 0)
        def _():
            run(nch0 // unit)

        @pl.when(cid != 0)
        def _():
            run(nch1 // unit)

        plsc.subcore_barrier()
        pltpu.sync_copy(acc.at[rsl], out.at[cid, sid])

    return body


@functools.lru_cache(maxsize=None)
def _tc_dinv(np_):
    def body(deg_ref, out_ref):
        d = deg_ref[0] + deg_ref[1] + 1.0
        out_ref[...] = lax.rsqrt(d)
    return pl.pallas_call(
        body,
        out_shape=jax.ShapeDtypeStruct((np_, DW), jnp.float32),
    )


@functools.lru_cache(maxsize=None)
def _tc_mm_scale(np_, d, br):
    grid = (np_ // br,)

    def body(x_ref, w_ref, dinv_ref, out_ref):
        h = jnp.dot(x_ref[...], w_ref[...],
                    preferred_element_type=jnp.float32)
        out_ref[...] = h * dinv_ref[...]

    return pl.pallas_call(
        body,
        grid=grid,
        in_specs=[
            pl.BlockSpec((br, d), lambda i: (i, 0)),
            pl.BlockSpec((d, d), lambda i: (0, 0)),
            pl.BlockSpec((br, 1), lambda i: (i, 0)),
        ],
        out_specs=pl.BlockSpec((br, d), lambda i: (i, 0)),
        out_shape=jax.ShapeDtypeStruct((np_, d), jnp.float32),
    )


@functools.lru_cache(maxsize=None)
def _tc_layer(np_, d, n, br):
    grid = (np_ // br,)

    def body(p0_ref, p1_ref, hs_ref, dinv_ref, b_ref, w_ref, out_ref):
        dinv = dinv_ref[...]
        a = (p0_ref[...] + p1_ref[...] + hs_ref[...]) * dinv + b_ref[...]
        rows = (lax.broadcasted_iota(jnp.int32, (br, 1), 0)
                + pl.program_id(0) * br)
        a = jnp.where(rows < n, jnp.maximum(a, 0.0), 0.0)
        out_ref[...] = jnp.dot(a, w_ref[...],
                               preferred_element_type=jnp.float32) * dinv

    return pl.pallas_call(
        body,
        grid=grid,
        in_specs=[
            pl.BlockSpec((br, d), lambda i: (i, 0)),
            pl.BlockSpec((br, d), lambda i: (i, 0)),
            pl.BlockSpec((br, d), lambda i: (i, 0)),
            pl.BlockSpec((br, 1), lambda i: (i, 0)),
            pl.BlockSpec((1, d), lambda i: (0, 0)),
            pl.BlockSpec((d, d), lambda i: (0, 0)),
        ],
        out_specs=pl.BlockSpec((br, d), lambda i: (i, 0)),
        out_shape=jax.ShapeDtypeStruct((np_, d), jnp.float32),
    )


@functools.lru_cache(maxsize=None)
def _tc_final(np_, d, br):
    grid = (np_ // br,)

    def body(p0_ref, p1_ref, hs_ref, dinv_ref, b_ref, out_ref):
        out_ref[...] = ((p0_ref[...] + p1_ref[...] + hs_ref[...])
                        * dinv_ref[...] + b_ref[...])

    return pl.pallas_call(
        body,
        grid=grid,
        in_specs=[
            pl.BlockSpec((br, d), lambda i: (i, 0)),
            pl.BlockSpec((br, d), lambda i: (i, 0)),
            pl.BlockSpec((br, d), lambda i: (i, 0)),
            pl.BlockSpec((br, 1), lambda i: (i, 0)),
            pl.BlockSpec((1, d), lambda i: (0, 0)),
        ],
        out_specs=pl.BlockSpec((br, d), lambda i: (i, 0)),
        out_shape=jax.ShapeDtypeStruct((np_, d), jnp.float32),
    )


def kernel(x, edge_index, batch, W1, b1, W2, b2, W3, b3):
    n, d = x.shape
    e = edge_index.shape[1]
    np_ = ((n + 1 + 127) // 128) * 128
    eb = NW * CH * 2 * BG
    ep = ((e + eb - 1) // eb) * eb
    br = 1280 if np_ % 1280 == 0 else 128

    pad = ep - e
    src = edge_index[0]
    dst = edge_index[1]
    if pad:
        fill = jnp.full((pad,), n, jnp.int32)
        src = jnp.concatenate([src, fill])
        dst = jnp.concatenate([dst, fill])
    src = src.reshape(ep // CH, CH)
    dst = dst.reshape(ep // CH, CH)
    x_pad = jnp.pad(x, ((0, np_ - n), (0, 0)))

    degp = _sc_deg(np_, ep)(dst)
    dinv16 = _tc_dinv(np_)(degp.reshape(NC, np_, DW))
    dinv_col = dinv16[:, :1]

    hs = _tc_mm_scale(np_, d, br)(x_pad, W1, dinv_col)
    scat = _sc_scatter(np_, d, ep)
    layer = _tc_layer(np_, d, n, br)
    for w_next, b_cur in ((W2, b1), (W3, b2)):
        p = scat(hs, src, dst)
        hs = layer(p[0].reshape(np_, d), p[1].reshape(np_, d),
                   hs, dinv_col, b_cur.reshape(1, d), w_next)
    p = scat(hs, src, dst)
    out = _tc_final(np_, d, br)(p[0].reshape(np_, d), p[1].reshape(np_, d),
                                hs, dinv_col, b3.reshape(1, d))
    return out[:n]


# speedup vs baseline: 1.0668x; 1.0013x over previous
import functools

import jax
import jax.numpy as jnp
from jax import lax
from jax.experimental import pallas as pl
from jax.experimental.pallas import tpu as pltpu
from jax.experimental.pallas import tpu_sc as plsc

NC = 2
NS = 16
NW = NC * NS
CH = 128
DW = 16
NBUF = 4
BG = 8


def _mesh():
    return plsc.VectorSubcoreMesh(
        core_axis_name="c", subcore_axis_name="s",
        num_cores=NC, num_subcores=NS)


@functools.lru_cache(maxsize=None)
def _sc_deg(np_, ep):
    rpt = np_ // NS
    nch = ep // CH // NW

    @functools.partial(
        pl.kernel,
        out_type=jax.ShapeDtypeStruct((NC, NS, rpt, DW), jnp.float32),
        mesh=_mesh(),
        scratch_types=[
            pltpu.VMEM((16, DW), jnp.float32),
            pltpu.VMEM((CH, DW), jnp.float32),
            pltpu.VMEM((nch, CH), jnp.int32),
            pltpu.VMEM_SHARED((np_, DW), jnp.float32),
            pltpu.SemaphoreType.DMA,
        ] + [pltpu.SemaphoreType.DMA for _ in range(NBUF)],
    )
    def body(dst2, out, zbuf, ones, dall, acc, zsem, *ssem):
        cid = lax.axis_index("c")
        sid = lax.axis_index("s")
        wid = sid * NC + cid
        z16 = jnp.zeros((16,), jnp.float32)
        o16 = jnp.ones((16,), jnp.float32)
        for i in range(16):
            zbuf[i, pl.ds(0, 16)] = z16
        for i in range(CH):
            ones[i, pl.ds(0, 16)] = o16
        for k in range(rpt // 16):
            pltpu.async_copy(zbuf, acc.at[pl.ds(sid * rpt + k * 16, 16)], zsem)
        pltpu.sync_copy(dst2.at[pl.ds(wid * nch, nch)], dall)
        for k in range(rpt // 16):
            pltpu.make_async_copy(
                zbuf, acc.at[pl.ds(sid * rpt + k * 16, 16)], zsem).wait()
        plsc.subcore_barrier()

        def scat(i, b):
            pltpu.async_copy(ones, acc.at[dall.at[i]], ssem[b], add=True)

        def sdrain(b):
            pltpu.make_async_copy(ones, acc.at[dall.at[0]], ssem[b]).wait()

        for b in range(NBUF):
            scat(b, b)

        def eloop(g, c):
            for b in range(NBUF):
                sdrain(b)
                scat(g * NBUF + b, b)
            return c
        lax.fori_loop(1, nch // NBUF, eloop, 0)
        for b in range(NBUF):
            sdrain(b)
        plsc.subcore_barrier()
        pltpu.sync_copy(acc.at[pl.ds(sid * rpt, rpt)], out.at[cid, sid])

    return body


@functools.lru_cache(maxsize=None)
def _sc_scatter(np_, d, ep):
    rpt = np_ // NS
    ncht = ep // CH
    unit = 2 * BG
    nch_t = ncht // NW

    @functools.partial(
        pl.kernel,
        out_type=jax.ShapeDtypeStruct((NC, NS, rpt, d), jnp.float32),
        mesh=_mesh(),
        scratch_types=[
            pltpu.VMEM((16, d), jnp.float32),
            pltpu.VMEM((BG, CH), jnp.int32),
            pltpu.VMEM((BG, CH), jnp.int32),
            pltpu.VMEM((BG, CH), jnp.int32),
            pltpu.VMEM((BG, CH), jnp.int32),
            pltpu.VMEM((CH, d), jnp.float32),
            pltpu.VMEM((CH, d), jnp.float32),
            pltpu.VMEM_SHARED((np_, d), jnp.float32),
        ] + [pltpu.SemaphoreType.DMA for _ in range(7)],
    )
    def body(hs, src2, dst2, out, zbuf, sb0, sb1, db0, db1, r0, r1,
             acc, zsem, i0, i1, g0, g1, s0, s1):
        sblk, dblk, rows = [sb0, sb1], [db0, db1], [r0, r1]
        isem, gsem, ssem = [i0, i1], [g0, g1], [s0, s1]
        cid = lax.axis_index("c")
        sid = lax.axis_index("s")
        cbase = (sid * NC + cid) * nch_t
        z16 = jnp.zeros((16,), jnp.float32)
        for i in range(16):
            for j in range(d // 16):
                zbuf[i, pl.ds(j * 16, 16)] = z16
        for k in range(rpt // 16):
            pltpu.async_copy(zbuf, acc.at[pl.ds(sid * rpt + k * 16, 16)], zsem)

        rsl = pl.ds(sid * rpt, rpt)

        def ifetch(kb, p):
            off = pl.multiple_of(cbase + kb * BG, 8)
            pltpu.async_copy(src2.at[pl.ds(off, BG)], sblk[p], isem[p])
            pltpu.async_copy(dst2.at[pl.ds(off, BG)], dblk[p], isem[p])

        def idrain(p):
            pltpu.make_async_copy(src2.at[pl.ds(0, BG)], sblk[p],
                                  isem[p]).wait()
            pltpu.make_async_copy(dst2.at[pl.ds(0, BG)], dblk[p],
                                  isem[p]).wait()

        def block(p, kb_next):
            def gath(j, b):
                pltpu.async_copy(hs.at[sblk[p].at[j]], rows[b], gsem[b])

            def gdrain(b):
                pltpu.make_async_copy(hs.at[sblk[p].at[0]], rows[b],
                                      gsem[b]).wait()

            def scat(j, b):
                pltpu.async_copy(rows[b], acc.at[dblk[p].at[j]], ssem[b],
                                 add=True)

            def sdrain(b):
                pltpu.make_async_copy(rows[b], acc.at[dblk[p].at[0]],
                                      ssem[b]).wait()

            idrain(p)
            gath(0, 0)
            gath(1, 1)
            if kb_next is not None:
                ifetch(kb_next, 1 - p)
            for j in range(BG):
                b = j % 2
                gdrain(b)
                scat(j, b)
                if j + 2 < BG:
                    sdrain(b)
                    gath(j + 2, b)
            sdrain(0)
            sdrain(1)

        ifetch(0, 0)
        for k in range(rpt // 16):
            pltpu.make_async_copy(
                zbuf, acc.at[pl.ds(sid * rpt + k * 16, 16)], zsem).wait()
        plsc.subcore_barrier()

        def pair(kp, c):
            block(0, kp * 2 + 1)
            block(1, kp * 2 + 2)
            return c

        npair = nch_t // unit
        lax.fori_loop(0, npair - 1, pair, 0)
        block(0, 2 * npair - 1)
        block(1, None)
        plsc.subcore_barrier()
        pltpu.sync_copy(acc.at[rsl], out.at[cid, sid])

    return body


@functools.lru_cache(maxsize=None)
def _tc_dinv(np_):
    def body(deg_ref, out_ref):
        d = deg_ref[0] + deg_ref[1] + 1.0
        out_ref[...] = lax.rsqrt(d)
    return pl.pallas_call(
        body,
        out_shape=jax.ShapeDtypeStruct((np_, DW), jnp.float32),
    )


@functools.lru_cache(maxsize=None)
def _tc_mm_scale(np_, d, br):
    grid = (np_ // br,)

    def body(x_ref, w_ref, dinv_ref, out_ref):
        h = jnp.dot(x_ref[...], w_ref[...],
                    preferred_element_type=jnp.float32)
        out_ref[...] = h * dinv_ref[...]

    return pl.pallas_call(
        body,
        grid=grid,
        in_specs=[
            pl.BlockSpec((br, d), lambda i: (i, 0)),
            pl.BlockSpec((d, d), lambda i: (0, 0)),
            pl.BlockSpec((br, 1), lambda i: (i, 0)),
        ],
        out_specs=pl.BlockSpec((br, d), lambda i: (i, 0)),
        out_shape=jax.ShapeDtypeStruct((np_, d), jnp.float32),
    )


@functools.lru_cache(maxsize=None)
def _tc_layer(np_, d, n, br):
    grid = (np_ // br,)

    def body(p0_ref, p1_ref, hs_ref, dinv_ref, b_ref, w_ref, out_ref):
        dinv = dinv_ref[...]
        a = (p0_ref[...] + p1_ref[...] + hs_ref[...]) * dinv + b_ref[...]
        rows = (lax.broadcasted_iota(jnp.int32, (br, 1), 0)
                + pl.program_id(0) * br)
        a = jnp.where(rows < n, jnp.maximum(a, 0.0), 0.0)
        out_ref[...] = jnp.dot(a, w_ref[...],
                               preferred_element_type=jnp.float32) * dinv

    return pl.pallas_call(
        body,
        grid=grid,
        in_specs=[
            pl.BlockSpec((br, d), lambda i: (i, 0)),
            pl.BlockSpec((br, d), lambda i: (i, 0)),
            pl.BlockSpec((br, d), lambda i: (i, 0)),
            pl.BlockSpec((br, 1), lambda i: (i, 0)),
            pl.BlockSpec((1, d), lambda i: (0, 0)),
            pl.BlockSpec((d, d), lambda i: (0, 0)),
        ],
        out_specs=pl.BlockSpec((br, d), lambda i: (i, 0)),
        out_shape=jax.ShapeDtypeStruct((np_, d), jnp.float32),
    )


@functools.lru_cache(maxsize=None)
def _tc_final(np_, d, br):
    grid = (np_ // br,)

    def body(p0_ref, p1_ref, hs_ref, dinv_ref, b_ref, out_ref):
        out_ref[...] = ((p0_ref[...] + p1_ref[...] + hs_ref[...])
                        * dinv_ref[...] + b_ref[...])

    return pl.pallas_call(
        body,
        grid=grid,
        in_specs=[
            pl.BlockSpec((br, d), lambda i: (i, 0)),
            pl.BlockSpec((br, d), lambda i: (i, 0)),
            pl.BlockSpec((br, d), lambda i: (i, 0)),
            pl.BlockSpec((br, 1), lambda i: (i, 0)),
            pl.BlockSpec((1, d), lambda i: (0, 0)),
        ],
        out_specs=pl.BlockSpec((br, d), lambda i: (i, 0)),
        out_shape=jax.ShapeDtypeStruct((np_, d), jnp.float32),
    )


def kernel(x, edge_index, batch, W1, b1, W2, b2, W3, b3):
    n, d = x.shape
    e = edge_index.shape[1]
    np_ = ((n + 1 + 127) // 128) * 128
    eb = NW * CH * 2 * BG
    ep = ((e + eb - 1) // eb) * eb
    br = 1280 if np_ % 1280 == 0 else 128

    pad = ep - e
    src = edge_index[0]
    dst = edge_index[1]
    if pad:
        fill = jnp.full((pad,), n, jnp.int32)
        src = jnp.concatenate([src, fill])
        dst = jnp.concatenate([dst, fill])
    src = src.reshape(ep // CH, CH)
    dst = dst.reshape(ep // CH, CH)
    x_pad = jnp.pad(x, ((0, np_ - n), (0, 0)))

    degp = _sc_deg(np_, ep)(dst)
    dinv16 = _tc_dinv(np_)(degp.reshape(NC, np_, DW))
    dinv_col = dinv16[:, :1]

    hs = _tc_mm_scale(np_, d, br)(x_pad, W1, dinv_col)
    scat = _sc_scatter(np_, d, ep)
    layer = _tc_layer(np_, d, n, br)
    for w_next, b_cur in ((W2, b1), (W3, b2)):
        p = scat(hs, src, dst)
        hs = layer(p[0].reshape(np_, d), p[1].reshape(np_, d),
                   hs, dinv_col, b_cur.reshape(1, d), w_next)
    p = scat(hs, src, dst)
    out = _tc_final(np_, d, br)(p[0].reshape(np_, d), p[1].reshape(np_, d),
                                hs, dinv_col, b3.reshape(1, d))
    return out[:n]


# speedup vs baseline: 1.2480x; 1.1698x over previous
import functools

import jax
import jax.numpy as jnp
from jax import lax
from jax.experimental import pallas as pl
from jax.experimental.pallas import tpu as pltpu
from jax.experimental.pallas import tpu_sc as plsc

NC = 2
NS = 16
NW = NC * NS
CH = 128
DW = 16
NBUF = 4
BG = 8


def _mesh():
    return plsc.VectorSubcoreMesh(
        core_axis_name="c", subcore_axis_name="s",
        num_cores=NC, num_subcores=NS)


@functools.lru_cache(maxsize=None)
def _sc_deg(np_, ep):
    rpt = np_ // NS
    nch = ep // CH // NW

    @functools.partial(
        pl.kernel,
        out_type=jax.ShapeDtypeStruct((NC, NS, rpt, DW), jnp.float32),
        mesh=_mesh(),
        scratch_types=[
            pltpu.VMEM((16, DW), jnp.float32),
            pltpu.VMEM((CH, DW), jnp.float32),
            pltpu.VMEM((nch, CH), jnp.int32),
            pltpu.VMEM_SHARED((np_, DW), jnp.float32),
            pltpu.SemaphoreType.DMA,
        ] + [pltpu.SemaphoreType.DMA for _ in range(NBUF)],
    )
    def body(dst2, out, zbuf, ones, dall, acc, zsem, *ssem):
        cid = lax.axis_index("c")
        sid = lax.axis_index("s")
        wid = sid * NC + cid
        z16 = jnp.zeros((16,), jnp.float32)
        o16 = jnp.ones((16,), jnp.float32)
        for i in range(16):
            zbuf[i, pl.ds(0, 16)] = z16
        for i in range(CH):
            ones[i, pl.ds(0, 16)] = o16
        for k in range(rpt // 16):
            pltpu.async_copy(zbuf, acc.at[pl.ds(sid * rpt + k * 16, 16)], zsem)
        pltpu.sync_copy(dst2.at[wid], dall)
        for k in range(rpt // 16):
            pltpu.make_async_copy(
                zbuf, acc.at[pl.ds(sid * rpt + k * 16, 16)], zsem).wait()
        plsc.subcore_barrier()

        def scat(i, b):
            pltpu.async_copy(ones, acc.at[dall.at[i]], ssem[b], add=True)

        def sdrain(b):
            pltpu.make_async_copy(ones, acc.at[dall.at[0]], ssem[b]).wait()

        for b in range(NBUF):
            scat(b, b)

        def eloop(g, c):
            for b in range(NBUF):
                sdrain(b)
                scat(g * NBUF + b, b)
            return c
        lax.fori_loop(1, nch // NBUF, eloop, 0)
        for b in range(NBUF):
            sdrain(b)
        plsc.subcore_barrier()
        pltpu.sync_copy(acc.at[pl.ds(sid * rpt, rpt)], out.at[cid, sid])

    return body


@functools.lru_cache(maxsize=None)
def _sc_scatter(np_, d, ep):
    rpt = np_ // NS
    ncht = ep // CH
    unit = 2 * BG
    nch_t = ncht // NW

    @functools.partial(
        pl.kernel,
        out_type=jax.ShapeDtypeStruct((NC, NS, rpt, d), jnp.float32),
        mesh=_mesh(),
        scratch_types=[
            pltpu.VMEM((16, d), jnp.float32),
            pltpu.VMEM((BG, CH), jnp.int32),
            pltpu.VMEM((BG, CH), jnp.int32),
            pltpu.VMEM((BG, CH), jnp.int32),
            pltpu.VMEM((BG, CH), jnp.int32),
            pltpu.VMEM((CH, d), jnp.float32),
            pltpu.VMEM((CH, d), jnp.float32),
            pltpu.VMEM_SHARED((np_, d), jnp.float32),
        ] + [pltpu.SemaphoreType.DMA for _ in range(7)],
    )
    def body(hs, src2, dst2, out, zbuf, sb0, sb1, db0, db1, r0, r1,
             acc, zsem, i0, i1, g0, g1, s0, s1):
        sblk, dblk, rows = [sb0, sb1], [db0, db1], [r0, r1]
        isem, gsem, ssem = [i0, i1], [g0, g1], [s0, s1]
        cid = lax.axis_index("c")
        sid = lax.axis_index("s")
        wid = sid * NC + cid
        z16 = jnp.zeros((16,), jnp.float32)
        for i in range(16):
            for j in range(d // 16):
                zbuf[i, pl.ds(j * 16, 16)] = z16
        for k in range(rpt // 16):
            pltpu.async_copy(zbuf, acc.at[pl.ds(sid * rpt + k * 16, 16)], zsem)

        rsl = pl.ds(sid * rpt, rpt)

        def ifetch(kb, p):
            pltpu.async_copy(src2.at[wid, pl.ds(kb * BG, BG)], sblk[p],
                             isem[p])
            pltpu.async_copy(dst2.at[wid, pl.ds(kb * BG, BG)], dblk[p],
                             isem[p])

        def idrain(p):
            pltpu.make_async_copy(src2.at[wid, pl.ds(0, BG)], sblk[p],
                                  isem[p]).wait()
            pltpu.make_async_copy(dst2.at[wid, pl.ds(0, BG)], dblk[p],
                                  isem[p]).wait()

        def block(p, kb_next):
            def gath(j, b):
                pltpu.async_copy(hs.at[sblk[p].at[j]], rows[b], gsem[b])

            def gdrain(b):
                pltpu.make_async_copy(hs.at[sblk[p].at[0]], rows[b],
                                      gsem[b]).wait()

            def scat(j, b):
                pltpu.async_copy(rows[b], acc.at[dblk[p].at[j]], ssem[b],
                                 add=True)

            def sdrain(b):
                pltpu.make_async_copy(rows[b], acc.at[dblk[p].at[0]],
                                      ssem[b]).wait()

            idrain(p)
            gath(0, 0)
            gath(1, 1)
            if kb_next is not None:
                ifetch(kb_next, 1 - p)
            for j in range(BG):
                b = j % 2
                gdrain(b)
                scat(j, b)
                if j + 2 < BG:
                    sdrain(b)
                    gath(j + 2, b)
            sdrain(0)
            sdrain(1)

        ifetch(0, 0)
        for k in range(rpt // 16):
            pltpu.make_async_copy(
                zbuf, acc.at[pl.ds(sid * rpt + k * 16, 16)], zsem).wait()
        plsc.subcore_barrier()

        def pair(kp, c):
            block(0, kp * 2 + 1)
            block(1, kp * 2 + 2)
            return c

        npair = nch_t // unit
        lax.fori_loop(0, npair - 1, pair, 0)
        block(0, 2 * npair - 1)
        block(1, None)
        plsc.subcore_barrier()
        pltpu.sync_copy(acc.at[rsl], out.at[cid, sid])

    return body


@functools.lru_cache(maxsize=None)
def _tc_dinv(np_):
    def body(deg_ref, out_ref):
        d = deg_ref[0] + deg_ref[1] + 1.0
        out_ref[...] = lax.rsqrt(d)
    return pl.pallas_call(
        body,
        out_shape=jax.ShapeDtypeStruct((np_, DW), jnp.float32),
    )


@functools.lru_cache(maxsize=None)
def _tc_mm_scale(np_, d, br):
    grid = (np_ // br,)

    def body(x_ref, w_ref, dinv_ref, out_ref):
        h = jnp.dot(x_ref[...], w_ref[...],
                    preferred_element_type=jnp.float32)
        out_ref[...] = h * dinv_ref[...]

    return pl.pallas_call(
        body,
        grid=grid,
        in_specs=[
            pl.BlockSpec((br, d), lambda i: (i, 0)),
            pl.BlockSpec((d, d), lambda i: (0, 0)),
            pl.BlockSpec((br, 1), lambda i: (i, 0)),
        ],
        out_specs=pl.BlockSpec((br, d), lambda i: (i, 0)),
        out_shape=jax.ShapeDtypeStruct((np_, d), jnp.float32),
    )


@functools.lru_cache(maxsize=None)
def _tc_layer(np_, d, n, br):
    grid = (np_ // br,)

    def body(p0_ref, p1_ref, hs_ref, dinv_ref, b_ref, w_ref, out_ref):
        dinv = dinv_ref[...]
        a = (p0_ref[...] + p1_ref[...] + hs_ref[...]) * dinv + b_ref[...]
        rows = (lax.broadcasted_iota(jnp.int32, (br, 1), 0)
                + pl.program_id(0) * br)
        a = jnp.where(rows < n, jnp.maximum(a, 0.0), 0.0)
        out_ref[...] = jnp.dot(a, w_ref[...],
                               preferred_element_type=jnp.float32) * dinv

    return pl.pallas_call(
        body,
        grid=grid,
        in_specs=[
            pl.BlockSpec((br, d), lambda i: (i, 0)),
            pl.BlockSpec((br, d), lambda i: (i, 0)),
            pl.BlockSpec((br, d), lambda i: (i, 0)),
            pl.BlockSpec((br, 1), lambda i: (i, 0)),
            pl.BlockSpec((1, d), lambda i: (0, 0)),
            pl.BlockSpec((d, d), lambda i: (0, 0)),
        ],
        out_specs=pl.BlockSpec((br, d), lambda i: (i, 0)),
        out_shape=jax.ShapeDtypeStruct((np_, d), jnp.float32),
    )


@functools.lru_cache(maxsize=None)
def _tc_final(np_, d, br):
    grid = (np_ // br,)

    def body(p0_ref, p1_ref, hs_ref, dinv_ref, b_ref, out_ref):
        out_ref[...] = ((p0_ref[...] + p1_ref[...] + hs_ref[...])
                        * dinv_ref[...] + b_ref[...])

    return pl.pallas_call(
        body,
        grid=grid,
        in_specs=[
            pl.BlockSpec((br, d), lambda i: (i, 0)),
            pl.BlockSpec((br, d), lambda i: (i, 0)),
            pl.BlockSpec((br, d), lambda i: (i, 0)),
            pl.BlockSpec((br, 1), lambda i: (i, 0)),
            pl.BlockSpec((1, d), lambda i: (0, 0)),
        ],
        out_specs=pl.BlockSpec((br, d), lambda i: (i, 0)),
        out_shape=jax.ShapeDtypeStruct((np_, d), jnp.float32),
    )


def kernel(x, edge_index, batch, W1, b1, W2, b2, W3, b3):
    n, d = x.shape
    e = edge_index.shape[1]
    np_ = ((n + 1 + 127) // 128) * 128
    eb = NW * CH * 2 * BG
    ep = ((e + eb - 1) // eb) * eb
    br = 1280 if np_ % 1280 == 0 else 128

    pad = ep - e
    src = edge_index[0]
    dst = edge_index[1]
    if pad:
        fill = jnp.full((pad,), n, jnp.int32)
        src = jnp.concatenate([src, fill])
        dst = jnp.concatenate([dst, fill])
    src = src.reshape(NW, ep // CH // NW, CH)
    dst = dst.reshape(NW, ep // CH // NW, CH)
    x_pad = jnp.pad(x, ((0, np_ - n), (0, 0)))

    degp = _sc_deg(np_, ep)(dst)
    dinv16 = _tc_dinv(np_)(degp.reshape(NC, np_, DW))
    dinv_col = dinv16[:, :1]

    hs = _tc_mm_scale(np_, d, br)(x_pad, W1, dinv_col)
    scat = _sc_scatter(np_, d, ep)
    layer = _tc_layer(np_, d, n, br)
    for w_next, b_cur in ((W2, b1), (W3, b2)):
        p = scat(hs, src, dst)
        hs = layer(p[0].reshape(np_, d), p[1].reshape(np_, d),
                   hs, dinv_col, b_cur.reshape(1, d), w_next)
    p = scat(hs, src, dst)
    out = _tc_final(np_, d, br)(p[0].reshape(np_, d), p[1].reshape(np_, d),
                                hs, dinv_col, b3.reshape(1, d))
    return out[:n]
